# baseline (speedup 1.0000x reference)
import functools

import jax
import jax.numpy as jnp
from jax import lax
from jax.experimental import pallas as pl
from jax.experimental.pallas import tpu as pltpu
from jax.experimental.pallas import tpu_sc as plsc

_VOCAB = 30522
_HIDDEN = 768
_L = 16
_NV = _HIDDEN // _L
_EPS = 1e-12

_NC, _NS = 2, 16
_NW = _NC * _NS
_SEQ = 128
_BATCH = 1024
_TOK = _BATCH * _SEQ
_TPW = _TOK // _NW
_SPW = _TPW // _SEQ
_C = 64
_NCHUNK = _SEQ // _C
_G = 16
_KPS = _C // _G
_NBUF = 4


def _lane_allsum(v, perms):
    for perm in perms:
        v = v + v.at[perm].get(mode="promise_in_bounds")
    return v


def _newton_rsqrt(x):
    i = plsc.bitcast(x, jnp.int32)
    i = 0x5F3759DF - (i >> 1)
    y = plsc.bitcast(i, jnp.float32)
    for _ in range(3):
        y = y * (1.5 - 0.5 * x * y * y)
    return y


_mesh = plsc.VectorSubcoreMesh(core_axis_name="c", subcore_axis_name="s")


@functools.partial(
    pl.kernel,
    mesh=_mesh,
    compiler_params=pltpu.CompilerParams(needs_layout_passes=False),
    out_type=jax.ShapeDtypeStruct((_TOK, _HIDDEN), jnp.float32),
    scratch_types=(
        [pltpu.VMEM((_SPW, _NCHUNK, _C), jnp.int32),
         pltpu.VMEM((_C, _HIDDEN), jnp.float32),
         pltpu.VMEM((1, _HIDDEN), jnp.float32),
         pltpu.VMEM((1, _HIDDEN), jnp.float32),
         pltpu.VMEM((1, _HIDDEN), jnp.float32)]
        + [pltpu.VMEM((_G, _HIDDEN), jnp.float32) for _ in range(_NBUF)]
        + [pltpu.SemaphoreType.DMA for _ in range(2 * _NBUF)]
    ),
)
def _emb_kernel(word_hbm, idx_hbm, pos_hbm, typ_hbm, gam_hbm, bet_hbm,
                out_hbm, idx_l, bias_v, typ_v, gam_v, bet_v, *bufs_sems):
    bufs = bufs_sems[:_NBUF]
    gsem = bufs_sems[_NBUF:2 * _NBUF]
    ssem = bufs_sems[2 * _NBUF:]
    wid = lax.axis_index("s") * _NC + lax.axis_index("c")
    iota = lax.iota(jnp.int32, _L)
    perms = tuple(iota ^ k for k in (8, 4, 2, 1))

    pltpu.sync_copy(idx_hbm.at[wid], idx_l)
    pltpu.sync_copy(typ_hbm.at[pl.ds(0, 1)], typ_v)
    pltpu.sync_copy(gam_hbm, gam_v)
    pltpu.sync_copy(bet_hbm, bet_v)

    def compute_tile(buf):
        def row_body(r, poff):
            acc1 = jnp.zeros((_L,), jnp.float32)
            acc2 = jnp.zeros((_L,), jnp.float32)
            p = poff + r
            for j in range(_NV):
                sl = pl.ds(j * _L, _L)
                v = buf[r, sl] + bias_v[p, sl]
                buf[r, sl] = v
                acc1 = acc1 + v
                acc2 = acc2 + v * v
            m_vec = _lane_allsum(acc1, perms) * (1.0 / _HIDDEN)
            var = _lane_allsum(acc2, perms) * (1.0 / _HIDDEN) - m_vec * m_vec
            rs = _newton_rsqrt(var + _EPS)
            for j in range(_NV):
                sl = pl.ds(j * _L, _L)
                v = buf[r, sl]
                buf[r, sl] = ((v - m_vec) * rs * gam_v[0, sl]
                              + bet_v[0, sl])
            return poff
        return row_body

    for c in range(_NCHUNK):
        pltpu.sync_copy(pos_hbm.at[pl.ds(c * _C, _C)], bias_v)

        def add_typ(r, carry):
            for j in range(_NV):
                sl = pl.ds(j * _L, _L)
                bias_v[r, sl] = bias_v[r, sl] + typ_v[0, sl]
            return carry

        lax.fori_loop(0, _C, add_typ, 0)

        def seq_body(s, carry, c=c):
            for k in range(_KPS):
                pltpu.async_copy(
                    word_hbm.at[idx_l.at[s, c, pl.ds(k * _G, _G)]],
                    bufs[k], gsem[k]).wait()
                lax.fori_loop(0, _G, compute_tile(bufs[k]),
                              jnp.int32(k * _G))
                obase = wid * _TPW + s * _SEQ + c * _C + k * _G
                pltpu.sync_copy(bufs[k], out_hbm.at[pl.ds(obase, _G)])
            return carry

        lax.fori_loop(0, _SPW, seq_body, 0)


def kernel(input_tokens, word_table, pos_table, type_table, ln_gamma, ln_beta):
    idx = input_tokens.astype(jnp.int32).reshape(_NW, _SPW, _NCHUNK, _C)
    out = _emb_kernel(
        word_table, idx, pos_table, type_table,
        ln_gamma.reshape(1, _HIDDEN), ln_beta.reshape(1, _HIDDEN))
    return out.reshape(_BATCH, _SEQ, _HIDDEN)


# speedup vs baseline: 1.2084x; 1.2084x over previous
import functools

import jax
import jax.numpy as jnp
from jax import lax
from jax.experimental import pallas as pl
from jax.experimental.pallas import tpu as pltpu
from jax.experimental.pallas import tpu_sc as plsc

_VOCAB = 30522
_HIDDEN = 768
_L = 16
_NV = _HIDDEN // _L
_EPS = 1e-12

_NC, _NS = 2, 16
_NW = _NC * _NS
_SEQ = 128
_BATCH = 1024
_TOK = _BATCH * _SEQ
_TPW = _TOK // _NW
_SPW = _TPW // _SEQ
_C = 64
_NCHUNK = _SEQ // _C
_G = 16
_KPS = _C // _G
_NBUF = 4


def _lane_allsum(v, perms):
    for perm in perms:
        v = v + v.at[perm].get(mode="promise_in_bounds")
    return v


def _newton_rsqrt(x):
    i = plsc.bitcast(x, jnp.int32)
    i = 0x5F3759DF - (i >> 1)
    y = plsc.bitcast(i, jnp.float32)
    for _ in range(3):
        y = y * (1.5 - 0.5 * x * y * y)
    return y


_mesh = plsc.VectorSubcoreMesh(core_axis_name="c", subcore_axis_name="s")


@functools.partial(
    pl.kernel,
    mesh=_mesh,
    compiler_params=pltpu.CompilerParams(needs_layout_passes=False),
    out_type=jax.ShapeDtypeStruct((_TOK, _HIDDEN), jnp.float32),
    scratch_types=(
        [pltpu.VMEM((_SPW, _NCHUNK, _C), jnp.int32),
         pltpu.VMEM((_C, _HIDDEN), jnp.float32),
         pltpu.VMEM((1, _HIDDEN), jnp.float32),
         pltpu.VMEM((1, _HIDDEN), jnp.float32),
         pltpu.VMEM((1, _HIDDEN), jnp.float32)]
        + [pltpu.VMEM((_G, _HIDDEN), jnp.float32) for _ in range(_NBUF)]
        + [pltpu.SemaphoreType.DMA for _ in range(2 * _NBUF)]
    ),
)
def _emb_kernel(word_hbm, idx_hbm, pos_hbm, typ_hbm, gam_hbm, bet_hbm,
                out_hbm, idx_l, bias_v, typ_v, gam_v, bet_v, *bufs_sems):
    bufs = bufs_sems[:_NBUF]
    gsem = bufs_sems[_NBUF:2 * _NBUF]
    ssem = bufs_sems[2 * _NBUF:]
    wid = lax.axis_index("s") * _NC + lax.axis_index("c")
    iota = lax.iota(jnp.int32, _L)
    perms = tuple(iota ^ k for k in (8, 4, 2, 1))

    pltpu.sync_copy(idx_hbm.at[wid], idx_l)
    pltpu.sync_copy(typ_hbm.at[pl.ds(0, 1)], typ_v)
    pltpu.sync_copy(gam_hbm, gam_v)
    pltpu.sync_copy(bet_hbm, bet_v)

    def compute_tile(buf):
        def row_body(r, poff):
            acc1 = jnp.zeros((_L,), jnp.float32)
            acc2 = jnp.zeros((_L,), jnp.float32)
            p = poff + r
            for j in range(_NV):
                sl = pl.ds(j * _L, _L)
                v = buf[r, sl] + bias_v[p, sl]
                buf[r, sl] = v
                acc1 = acc1 + v
                acc2 = acc2 + v * v
            m_vec = _lane_allsum(acc1, perms) * (1.0 / _HIDDEN)
            var = _lane_allsum(acc2, perms) * (1.0 / _HIDDEN) - m_vec * m_vec
            rs = _newton_rsqrt(var + _EPS)
            for j in range(_NV):
                sl = pl.ds(j * _L, _L)
                v = buf[r, sl]
                buf[r, sl] = ((v - m_vec) * rs * gam_v[0, sl]
                              + bet_v[0, sl])
            return poff
        return row_body

    for c in range(_NCHUNK):
        pltpu.sync_copy(pos_hbm.at[pl.ds(c * _C, _C)], bias_v)

        def add_typ(r, carry):
            for j in range(_NV):
                sl = pl.ds(j * _L, _L)
                bias_v[r, sl] = bias_v[r, sl] + typ_v[0, sl]
            return carry

        lax.fori_loop(0, _C, add_typ, 0)

        for k in range(2):
            pltpu.async_copy(
                word_hbm.at[idx_l.at[0, c, pl.ds(k * _G, _G)]],
                bufs[k], gsem[k])

        def seq_body(s, carry, c=c):
            for k in range(_KPS):
                k2 = (k + 2) % _NBUF
                pltpu.make_async_copy(
                    word_hbm.at[pl.ds(0, _G)], bufs[k], gsem[k]).wait()
                lax.fori_loop(0, _G, compute_tile(bufs[k]),
                              jnp.int32(k * _G))
                obase = wid * _TPW + s * _SEQ + c * _C + k * _G
                pltpu.async_copy(
                    bufs[k], out_hbm.at[pl.ds(obase, _G)], ssem[k])
                if k < 2:
                    @pl.when(s >= 1)
                    def _():
                        pltpu.make_async_copy(
                            bufs[k2], out_hbm.at[pl.ds(0, _G)],
                            ssem[k2]).wait()
                    pltpu.async_copy(
                        word_hbm.at[idx_l.at[s, c, pl.ds(k2 * _G, _G)]],
                        bufs[k2], gsem[k2])
                else:
                    pltpu.make_async_copy(
                        bufs[k2], out_hbm.at[pl.ds(0, _G)],
                        ssem[k2]).wait()

                    @pl.when(s < _SPW - 1)
                    def _():
                        pltpu.async_copy(
                            word_hbm.at[idx_l.at[s + 1, c,
                                                 pl.ds(k2 * _G, _G)]],
                            bufs[k2], gsem[k2])
            return carry

        lax.fori_loop(0, _SPW, seq_body, 0)

        for k in (2, 3):
            pltpu.make_async_copy(
                bufs[k], out_hbm.at[pl.ds(0, _G)], ssem[k]).wait()


def kernel(input_tokens, word_table, pos_table, type_table, ln_gamma, ln_beta):
    idx = input_tokens.astype(jnp.int32).reshape(_NW, _SPW, _NCHUNK, _C)
    out = _emb_kernel(
        word_table, idx, pos_table, type_table,
        ln_gamma.reshape(1, _HIDDEN), ln_beta.reshape(1, _HIDDEN))
    return out.reshape(_BATCH, _SEQ, _HIDDEN)
